# baseline (speedup 1.0000x reference)
import functools
import math

import jax
import jax.numpy as jnp
from jax import lax
from jax.experimental import pallas as pl
from jax.experimental.pallas import tpu as pltpu
from jax.experimental.pallas import tpu_sc as plsc

N = 10000
E = 160000
IN_DIM = 128
OUT_DIM = 128
E_DIM = 16
TIME_DIM = 32
NUM_HEADS = 8
HEAD = OUT_DIM // NUM_HEADS
NUM_RELS = 8
NUM_NTYPES = 4
ROW = IN_DIM + 2 * TIME_DIM
EROW = 32
CDIM = 144

EBLK = 2048
EBLK_LOG2 = 11
NBLOCKS = E // EBLK + NUM_RELS
E_PAD = NBLOCKS * EBLK
E_OUT = E_PAD + 256
N_PAD = 10016
RB_PAD = 96

NC = 2
NS = 16
NW = NC * NS
EPW = E_PAD // NW
GCHUNK = 344
NCHUNKS = EPW // GCHUNK
NPT = N_PAD // NS

SHARE = E // NS
NVR = SHARE // 16
FILL = E_OUT // NS


@functools.cache
def _sc_mesh():
    return plsc.VectorSubcoreMesh(core_axis_name="c", subcore_axis_name="s",
                                  num_cores=NC, num_subcores=NS)


def _sort_body(src_hbm, dst_hbm, et_hbm, srcp_hbm, dstp_hbm, perm_hbm,
               relblk_hbm, et_v, src_v, dst_v, pos_v, pv_v, fill_v, cnt_v,
               allcnt_v, rb_v, cnt_sh, srcp_sh, dstp_sh, perm_sh, sem):
    cid = lax.axis_index("c")
    sid = lax.axis_index("s")

    @pl.when(cid == 0)
    def _():
        one = jnp.full((16,), 1, jnp.int32)
        lane = lax.broadcasted_iota(jnp.int32, (16,), 0)
        pltpu.sync_copy(et_hbm.at[pl.ds(sid * SHARE, SHARE)], et_v)

        def hist_step(i, accs):
            v = et_v[pl.ds(i * 16, 16)]
            return tuple(accs[r] + one - jnp.minimum(one, jnp.abs(v - r))
                         for r in range(NUM_RELS))

        accs = lax.fori_loop(
            0, NVR, hist_step,
            tuple(jnp.zeros((16,), jnp.int32) for _ in range(NUM_RELS)))
        counts = jnp.zeros((16,), jnp.int32)
        for r in range(NUM_RELS):
            eqv = one - jnp.minimum(one, jnp.abs(lane - r))
            counts = counts + eqv * jnp.sum(accs[r])
        cnt_v[...] = counts
        pltpu.sync_copy(cnt_v, cnt_sh.at[sid])
        plsc.subcore_barrier()

        pltpu.sync_copy(cnt_sh, allcnt_v)
        gl = jnp.zeros((16,), jnp.int32)
        base = jnp.zeros((16,), jnp.int32)
        sid_v = jnp.zeros((16,), jnp.int32) + sid
        for t in range(NS):
            row = allcnt_v[t]
            gl = gl + row
            ltv = jnp.minimum(one, jnp.maximum(one * 0, sid_v - t))
            base = base + ltv * row
        padded = ((gl + (EBLK - 1)) >> EBLK_LOG2) << EBLK_LOG2
        offs = plsc.cumsum(padded) - padded
        wb = []
        blkoff = []
        wbsum = offs + base
        for r in range(NUM_RELS):
            eqv = one - jnp.minimum(one, jnp.abs(lane - r))
            tmp = plsc.cummax(eqv * wbsum)
            wb.append(plsc.cummax(lax.rev(tmp, (0,))))
            blkoff.append(jnp.sum(eqv * offs) >> EBLK_LOG2)

        def zfill(i, c):
            fill_v[pl.ds(i * 16, 16)] = jnp.zeros((16,), jnp.int32)
            return c
        lax.fori_loop(0, FILL // 16, zfill, 0)
        pltpu.sync_copy(fill_v, srcp_sh.at[pl.ds(sid * FILL, FILL)])
        pltpu.sync_copy(fill_v, perm_sh.at[pl.ds(sid * FILL, FILL)])

        def nfill(i, c):
            fill_v[pl.ds(i * 16, 16)] = jnp.full((16,), N, jnp.int32)
            return c
        lax.fori_loop(0, FILL // 16, nfill, 0)
        pltpu.sync_copy(fill_v, dstp_sh.at[pl.ds(sid * FILL, FILL)])
        plsc.subcore_barrier()

        pltpu.sync_copy(src_hbm.at[pl.ds(sid * SHARE, SHARE)], src_v)
        pltpu.sync_copy(dst_hbm.at[pl.ds(sid * SHARE, SHARE)], dst_v)
        ebase = sid * SHARE

        def pos_step(i, wbs):
            v = et_v[pl.ds(i * 16, 16)]
            pos = jnp.zeros((16,), jnp.int32)
            out = []
            for r in range(NUM_RELS):
                m = one - jnp.minimum(one, jnp.abs(v - r))
                c = plsc.cumsum(m)
                cand = wbs[r] + c - 1
                pos = pos + m * (cand - pos)
                out.append(wbs[r] + plsc.cummax(lax.rev(c, (0,))))
            pos_v[pl.ds(i * 16, 16)] = pos
            pv_v[pl.ds(i * 16, 16)] = lane + (ebase + i * 16)
            return tuple(out)

        lax.fori_loop(0, NVR, pos_step, tuple(wb))
        c1 = pltpu.async_copy(src_v, srcp_sh.at[pos_v], sem)
        c2 = pltpu.async_copy(dst_v, dstp_sh.at[pos_v], sem)
        c3 = pltpu.async_copy(pv_v, perm_sh.at[pos_v], sem)
        c1.wait()
        c2.wait()
        c3.wait()
        plsc.subcore_barrier()
        pltpu.sync_copy(srcp_sh.at[pl.ds(sid * FILL, FILL)],
                        srcp_hbm.at[pl.ds(sid * FILL, FILL)])
        pltpu.sync_copy(dstp_sh.at[pl.ds(sid * FILL, FILL)],
                        dstp_hbm.at[pl.ds(sid * FILL, FILL)])
        pltpu.sync_copy(perm_sh.at[pl.ds(sid * FILL, FILL)],
                        perm_hbm.at[pl.ds(sid * FILL, FILL)])

        @pl.when(sid == 0)
        def _():
            for j in range(RB_PAD // 16):
                bi = lane + 16 * j
                acc = jnp.zeros((16,), jnp.int32)
                for r in range(1, NUM_RELS):
                    d = bi - blkoff[r] + 1
                    acc = acc + jnp.minimum(one, jnp.maximum(one * 0, d))
                rb_v[pl.ds(16 * j, 16)] = acc
            pltpu.sync_copy(rb_v, relblk_hbm)


@jax.jit
def _sort(src, dst, et):
    k = pl.kernel(
        _sort_body,
        out_type=(jax.ShapeDtypeStruct((E_OUT,), jnp.int32),
                  jax.ShapeDtypeStruct((E_OUT,), jnp.int32),
                  jax.ShapeDtypeStruct((E_OUT,), jnp.int32),
                  jax.ShapeDtypeStruct((RB_PAD,), jnp.int32)),
        mesh=_sc_mesh(),
        scratch_types=[
            pltpu.VMEM((SHARE,), jnp.int32),
            pltpu.VMEM((SHARE,), jnp.int32),
            pltpu.VMEM((SHARE,), jnp.int32),
            pltpu.VMEM((SHARE,), jnp.int32),
            pltpu.VMEM((SHARE,), jnp.int32),
            pltpu.VMEM((FILL,), jnp.int32),
            pltpu.VMEM((16,), jnp.int32),
            pltpu.VMEM((NS, 16), jnp.int32),
            pltpu.VMEM((RB_PAD,), jnp.int32),
            pltpu.VMEM_SHARED((NS, 16), jnp.int32),
            pltpu.VMEM_SHARED((E_OUT,), jnp.int32),
            pltpu.VMEM_SHARED((E_OUT,), jnp.int32),
            pltpu.VMEM_SHARED((E_OUT,), jnp.int32),
            pltpu.SemaphoreType.DMA,
        ],
        compiler_params=pltpu.CompilerParams(use_tc_tiling_on_sc=False,
                                             needs_layout_passes=False),
    )
    return k(src, dst, et)


def _gather_body(table_hbm, etab_hbm, srcidx_hbm, dstidx_hbm, permidx_hbm,
                 out_s_hbm, out_d_hbm, out_e_hbm, idx_s, idx_d, idx_p,
                 rows_s_v, rows_d_v, erows_v, sem):
    cid = lax.axis_index("c")
    sid = lax.axis_index("s")
    base = (cid * NS + sid) * EPW

    def step(j, carry):
        off = base + j * GCHUNK
        c1 = pltpu.async_copy(srcidx_hbm.at[pl.ds(off, GCHUNK)], idx_s, sem)
        c2 = pltpu.async_copy(dstidx_hbm.at[pl.ds(off, GCHUNK)], idx_d, sem)
        c3 = pltpu.async_copy(permidx_hbm.at[pl.ds(off, GCHUNK)], idx_p, sem)
        c1.wait()
        c2.wait()
        c3.wait()
        g1 = pltpu.async_copy(table_hbm.at[idx_s], rows_s_v, sem)
        g2 = pltpu.async_copy(table_hbm.at[idx_d], rows_d_v, sem)
        g3 = pltpu.async_copy(etab_hbm.at[idx_p], erows_v, sem)
        g1.wait()
        g2.wait()
        g3.wait()
        pltpu.sync_copy(rows_s_v, out_s_hbm.at[pl.ds(off, GCHUNK)])
        pltpu.sync_copy(rows_d_v, out_d_hbm.at[pl.ds(off, GCHUNK)])
        pltpu.sync_copy(erows_v, out_e_hbm.at[pl.ds(off, GCHUNK)])
        return carry

    lax.fori_loop(0, NCHUNKS, step, 0)


@jax.jit
def _gather(table, etab, srcp, dstp, perm):
    k = pl.kernel(
        _gather_body,
        out_type=(jax.ShapeDtypeStruct((E_PAD, ROW), jnp.bfloat16),
                  jax.ShapeDtypeStruct((E_PAD, ROW), jnp.bfloat16),
                  jax.ShapeDtypeStruct((E_PAD, EROW), jnp.float32)),
        mesh=_sc_mesh(),
        scratch_types=[
            pltpu.VMEM((GCHUNK,), jnp.int32),
            pltpu.VMEM((GCHUNK,), jnp.int32),
            pltpu.VMEM((GCHUNK,), jnp.int32),
            pltpu.VMEM((GCHUNK, ROW), jnp.bfloat16),
            pltpu.VMEM((GCHUNK, ROW), jnp.bfloat16),
            pltpu.VMEM((GCHUNK, EROW), jnp.float32),
            pltpu.SemaphoreType.DMA,
        ],
        compiler_params=pltpu.CompilerParams(use_tc_tiling_on_sc=False),
    )
    return k(table, etab, srcp, dstp, perm)


SCHUNK = 128
SNCHUNKS = EPW // SCHUNK


def _scatter_body(contrib_hbm, dstidx_hbm, zeros_hbm, out_hbm,
                  idx_v, rows_v, acc_sh):
    cid = lax.axis_index("c")
    sid = lax.axis_index("s")
    pltpu.sync_copy(zeros_hbm.at[pl.ds(sid * NPT, NPT)],
                    acc_sh.at[pl.ds(sid * NPT, NPT)])
    plsc.subcore_barrier()
    base = (cid * NS + sid) * EPW

    def step(j, carry):
        off = base + j * SCHUNK
        pltpu.sync_copy(dstidx_hbm.at[pl.ds(off, SCHUNK)], idx_v)
        pltpu.sync_copy(contrib_hbm.at[pl.ds(off, SCHUNK)], rows_v)
        pltpu.sync_copy(rows_v, acc_sh.at[idx_v], add=True)
        return carry

    lax.fori_loop(0, SNCHUNKS, step, 0)
    plsc.subcore_barrier()
    pltpu.sync_copy(acc_sh.at[pl.ds(sid * NPT, NPT)],
                    out_hbm.at[cid, pl.ds(sid * NPT, NPT)])


@jax.jit
def _scatter(contrib, dstp, zeros):
    k = pl.kernel(
        _scatter_body,
        out_type=jax.ShapeDtypeStruct((NC, N_PAD, CDIM), jnp.float32),
        mesh=_sc_mesh(),
        scratch_types=[
            pltpu.VMEM((SCHUNK,), jnp.int32),
            pltpu.VMEM((SCHUNK, CDIM), jnp.float32),
            pltpu.VMEM_SHARED((N_PAD, CDIM), jnp.float32),
        ],
        compiler_params=pltpu.CompilerParams(use_tc_tiling_on_sc=False),
    )
    return k(contrib, dstp, zeros)


def _layer_norm(x, g, b, eps=1e-5):
    mu = jnp.mean(x, axis=-1, keepdims=True)
    var = jnp.mean((x - mu) * (x - mu), axis=-1, keepdims=True)
    return (x - mu) * jax.lax.rsqrt(var + eps) * g + b


def _head_sum_mat():
    i = lax.broadcasted_iota(jnp.int32, (OUT_DIM, NUM_HEADS), 0)
    h = lax.broadcasted_iota(jnp.int32, (OUT_DIM, NUM_HEADS), 1)
    return (i // HEAD == h).astype(jnp.float32)


def _head_bcast_mat():
    h = lax.broadcasted_iota(jnp.int32, (NUM_HEADS, OUT_DIM), 0)
    i = lax.broadcasted_iota(jnp.int32, (NUM_HEADS, OUT_DIM), 1)
    return (i // HEAD == h).astype(jnp.float32)


def _edge_kernel_body(rb_ref, rs_ref, rd_ref, re_ref,
                      wq_ref, wk_ref, wv_ref, gs_ref, bs_ref, gd_ref, bd_ref,
                      out_ref):
    rs = rs_ref[...].astype(jnp.float32)
    rd = rd_ref[...].astype(jnp.float32)
    t = re_ref[:, E_DIM:E_DIM + 1]
    hs = rs[:, :IN_DIM]
    dia_s = jnp.sin(rs[:, IN_DIM:IN_DIM + TIME_DIM] * t
                    + rs[:, IN_DIM + TIME_DIM:]) * hs[:, :TIME_DIM]
    xs = jnp.concatenate([dia_s, hs[:, TIME_DIM:], re_ref[:, :E_DIM]], axis=1)
    xs = _layer_norm(xs, gs_ref[...], bs_ref[...])
    hd = rd[:, :IN_DIM]
    dia_d = jnp.sin(rd[:, IN_DIM:IN_DIM + TIME_DIM] * t
                    + rd[:, IN_DIM + TIME_DIM:]) * hd[:, :TIME_DIM]
    xd = jnp.concatenate([dia_d, hd[:, TIME_DIM:]], axis=1)
    xd = _layer_norm(xd, gd_ref[...], bd_ref[...])

    q = jnp.dot(xd.astype(jnp.bfloat16), wq_ref[0].astype(jnp.bfloat16),
                preferred_element_type=jnp.float32)
    k = jnp.dot(xs.astype(jnp.bfloat16), wk_ref[0].astype(jnp.bfloat16),
                preferred_element_type=jnp.float32)
    v = jnp.dot(xs.astype(jnp.bfloat16), wv_ref[0].astype(jnp.bfloat16),
                preferred_element_type=jnp.float32)

    attn = jnp.dot(q * k, _head_sum_mat(),
                   preferred_element_type=jnp.float32) * (1.0 / math.sqrt(OUT_DIM))
    ex = jnp.exp(attn)
    vw = v * jnp.dot(ex, _head_bcast_mat(),
                     preferred_element_type=jnp.float32)
    out_ref[...] = jnp.concatenate(
        [ex, jnp.zeros((ex.shape[0], CDIM - OUT_DIM - NUM_HEADS), jnp.float32), vw],
        axis=1)


@jax.jit
def _edge_compute(relblk, rows_s, rows_d, rows_e, Wq, Wk, Wv, gs, bs, gd, bd):
    grid_spec = pltpu.PrefetchScalarGridSpec(
        num_scalar_prefetch=1,
        grid=(NBLOCKS,),
        in_specs=[
            pl.BlockSpec((EBLK, ROW), lambda i, rb: (i, 0)),
            pl.BlockSpec((EBLK, ROW), lambda i, rb: (i, 0)),
            pl.BlockSpec((EBLK, EROW), lambda i, rb: (i, 0)),
            pl.BlockSpec((1, IN_DIM, OUT_DIM), lambda i, rb: (rb[i], 0, 0)),
            pl.BlockSpec((1, IN_DIM + E_DIM, OUT_DIM), lambda i, rb: (rb[i], 0, 0)),
            pl.BlockSpec((1, IN_DIM + E_DIM, OUT_DIM), lambda i, rb: (rb[i], 0, 0)),
            pl.BlockSpec((1, IN_DIM + E_DIM), lambda i, rb: (0, 0)),
            pl.BlockSpec((1, IN_DIM + E_DIM), lambda i, rb: (0, 0)),
            pl.BlockSpec((1, IN_DIM), lambda i, rb: (0, 0)),
            pl.BlockSpec((1, IN_DIM), lambda i, rb: (0, 0)),
        ],
        out_specs=pl.BlockSpec((EBLK, CDIM), lambda i, rb: (i, 0)),
    )
    return pl.pallas_call(
        _edge_kernel_body,
        grid_spec=grid_spec,
        out_shape=jax.ShapeDtypeStruct((E_PAD, CDIM), jnp.float32),
    )(relblk, rows_s, rows_d, rows_e, Wq, Wk, Wv, gs, bs, gd, bd)


NBLK = 1000


def _typed(x, et, w_ref, nrel):
    acc = jnp.zeros((x.shape[0], OUT_DIM), jnp.float32)
    for r in range(nrel):
        m = (et == r).astype(jnp.float32)
        acc = acc + m * jnp.dot(x, w_ref[r], preferred_element_type=jnp.float32)
    return acc


def _epi_kernel_body(p_ref, srch_ref, nt_ref, hb_ref, wa_ref, skip_ref, out_ref):
    s = p_ref[0] + p_ref[1]
    den = s[:, :NUM_HEADS]
    num = s[:, NUM_HEADS + (CDIM - OUT_DIM - NUM_HEADS):]
    den = jnp.where(den > 0.0, den, 1.0)
    h = num / jnp.dot(den, _head_bcast_mat(), preferred_element_type=jnp.float32)
    nt = nt_ref[...]
    onehot = (nt == lax.broadcasted_iota(jnp.int32, (1, NUM_NTYPES), 1)
              ).astype(jnp.float32)
    h = h + jnp.dot(onehot, hb_ref[...], preferred_element_type=jnp.float32)
    h2 = _typed(h, nt, wa_ref, NUM_NTYPES)
    al = jax.nn.sigmoid(jnp.dot(onehot, skip_ref[...],
                                preferred_element_type=jnp.float32))
    out_ref[...] = h2 * al + srch_ref[...] * (1.0 - al)


@jax.jit
def _epilogue(partials, src_h, nt_col, h_bias, Wa, skip_col):
    grid = (N // NBLK,)
    return pl.pallas_call(
        _epi_kernel_body,
        grid=grid,
        in_specs=[
            pl.BlockSpec((NC, NBLK, CDIM), lambda i: (0, i, 0)),
            pl.BlockSpec((NBLK, IN_DIM), lambda i: (i, 0)),
            pl.BlockSpec((NBLK, 1), lambda i: (i, 0)),
            pl.BlockSpec((NUM_NTYPES, OUT_DIM), lambda i: (0, 0)),
            pl.BlockSpec((NUM_NTYPES, OUT_DIM, OUT_DIM), lambda i: (0, 0, 0)),
            pl.BlockSpec((NUM_NTYPES, 1), lambda i: (0, 0)),
        ],
        out_specs=pl.BlockSpec((NBLK, OUT_DIM), lambda i: (i, 0)),
        out_shape=jax.ShapeDtypeStruct((N, OUT_DIM), jnp.float32),
    )(partials, src_h, nt_col, h_bias, Wa, skip_col)


def kernel(src_h, src_tw, src_tb, edge_h, edge_date, Wq, Wk, Wv, Wa,
           src_ln_g, src_ln_b, dst_ln_g, dst_ln_b, h_bias, skip,
           edge_index, edge_etype, dst_ntype):
    src = edge_index[0]
    dst = edge_index[1]
    table = jnp.pad(jnp.concatenate([src_h, src_tw, src_tb], axis=1),
                    ((0, N_PAD - N), (0, 0))).astype(jnp.bfloat16)
    etab = jnp.concatenate(
        [edge_h, edge_date.reshape(E, 1),
         jnp.zeros((E, EROW - E_DIM - 1), jnp.float32)], axis=1)
    srcp, dstp, perm, relblk = _sort(src, dst, edge_etype)
    rows_s, rows_d, rows_e = _gather(table, etab, srcp, dstp, perm)
    contrib = _edge_compute(
        relblk, rows_s, rows_d, rows_e, Wq, Wk, Wv,
        src_ln_g.reshape(1, -1), src_ln_b.reshape(1, -1),
        dst_ln_g.reshape(1, -1), dst_ln_b.reshape(1, -1))
    partials = _scatter(contrib, dstp, jnp.zeros((N_PAD, CDIM), jnp.float32))
    return _epilogue(partials, src_h, dst_ntype.reshape(N, 1),
                     h_bias, Wa, skip.reshape(NUM_NTYPES, 1))


# speedup vs baseline: 1.0226x; 1.0226x over previous
import functools
import math

import jax
import jax.numpy as jnp
from jax import lax
from jax.experimental import pallas as pl
from jax.experimental.pallas import tpu as pltpu
from jax.experimental.pallas import tpu_sc as plsc

N = 10000
E = 160000
IN_DIM = 128
OUT_DIM = 128
E_DIM = 16
TIME_DIM = 32
NUM_HEADS = 8
HEAD = OUT_DIM // NUM_HEADS
NUM_RELS = 8
NUM_NTYPES = 4
ROW = IN_DIM + 2 * TIME_DIM
CDIM = 144

NC = 2
NS = 16
NW = NC * NS
EPW = E // NW
GCHUNK = 200
NCHUNKS = EPW // GCHUNK
NPT = N // NS

@functools.cache
def _sc_mesh():
    return plsc.VectorSubcoreMesh(core_axis_name="c", subcore_axis_name="s",
                                  num_cores=NC, num_subcores=NS)


def _gather_body(table_hbm, srcidx_hbm, dstidx_hbm, out_s_hbm, out_d_hbm,
                 idx_v, rows_v, sem):
    cid = lax.axis_index("c")
    sid = lax.axis_index("s")
    base = (cid * NS + sid) * EPW

    def step(j, carry):
        off = base + j * GCHUNK
        pltpu.sync_copy(srcidx_hbm.at[pl.ds(off, GCHUNK)], idx_v)
        pltpu.async_copy(table_hbm.at[idx_v], rows_v, sem).wait()
        pltpu.sync_copy(rows_v, out_s_hbm.at[pl.ds(off, GCHUNK)])
        pltpu.sync_copy(dstidx_hbm.at[pl.ds(off, GCHUNK)], idx_v)
        pltpu.async_copy(table_hbm.at[idx_v], rows_v, sem).wait()
        pltpu.sync_copy(rows_v, out_d_hbm.at[pl.ds(off, GCHUNK)])
        return carry

    lax.fori_loop(0, NCHUNKS, step, 0)


@jax.jit
def _gather(table, src, dst):
    k = pl.kernel(
        _gather_body,
        out_type=(jax.ShapeDtypeStruct((E, ROW), jnp.bfloat16),
                  jax.ShapeDtypeStruct((E, ROW), jnp.bfloat16)),
        mesh=_sc_mesh(),
        scratch_types=[
            pltpu.VMEM((GCHUNK,), jnp.int32),
            pltpu.VMEM((GCHUNK, ROW), jnp.bfloat16),
            pltpu.SemaphoreType.DMA,
        ],
        compiler_params=pltpu.CompilerParams(use_tc_tiling_on_sc=False),
    )
    return k(table, src, dst)


def _scatter_body(contrib_hbm, dstidx_hbm, zeros_hbm, out_hbm,
                  idx_v, rows_v, acc_sh):
    cid = lax.axis_index("c")
    sid = lax.axis_index("s")
    pltpu.sync_copy(zeros_hbm.at[pl.ds(sid * NPT, NPT)],
                    acc_sh.at[pl.ds(sid * NPT, NPT)])
    plsc.subcore_barrier()
    base = (cid * NS + sid) * EPW

    def step(j, carry):
        off = base + j * GCHUNK
        pltpu.sync_copy(dstidx_hbm.at[pl.ds(off, GCHUNK)], idx_v)
        pltpu.sync_copy(contrib_hbm.at[pl.ds(off, GCHUNK)], rows_v)
        pltpu.sync_copy(rows_v, acc_sh.at[idx_v], add=True)
        return carry

    lax.fori_loop(0, NCHUNKS, step, 0)
    plsc.subcore_barrier()
    pltpu.sync_copy(acc_sh.at[pl.ds(sid * NPT, NPT)],
                    out_hbm.at[cid, pl.ds(sid * NPT, NPT)])


@jax.jit
def _scatter(contrib, dst, zeros):
    k = pl.kernel(
        _scatter_body,
        out_type=jax.ShapeDtypeStruct((NC, N, CDIM), jnp.float32),
        mesh=_sc_mesh(),
        scratch_types=[
            pltpu.VMEM((GCHUNK,), jnp.int32),
            pltpu.VMEM((GCHUNK, CDIM), jnp.float32),
            pltpu.VMEM_SHARED((N, CDIM), jnp.float32),
        ],
        compiler_params=pltpu.CompilerParams(use_tc_tiling_on_sc=False),
    )
    return k(contrib, dst, zeros)


EBLK = 2000


def _layer_norm(x, g, b, eps=1e-5):
    mu = jnp.mean(x, axis=-1, keepdims=True)
    var = jnp.mean((x - mu) * (x - mu), axis=-1, keepdims=True)
    return (x - mu) * jax.lax.rsqrt(var + eps) * g + b


def _typed(x, et, w_ref, nrel, dtype=jnp.float32):
    acc = jnp.zeros((x.shape[0], OUT_DIM), jnp.float32)
    xc = x.astype(dtype)
    for r in range(nrel):
        m = (et == r).astype(jnp.float32)
        acc = acc + m * jnp.dot(xc, w_ref[r].astype(dtype),
                                preferred_element_type=jnp.float32)
    return acc


def _head_sum_mat():
    i = lax.broadcasted_iota(jnp.int32, (OUT_DIM, NUM_HEADS), 0)
    h = lax.broadcasted_iota(jnp.int32, (OUT_DIM, NUM_HEADS), 1)
    return (i // HEAD == h).astype(jnp.float32)


def _head_bcast_mat():
    h = lax.broadcasted_iota(jnp.int32, (NUM_HEADS, OUT_DIM), 0)
    i = lax.broadcasted_iota(jnp.int32, (NUM_HEADS, OUT_DIM), 1)
    return (i // HEAD == h).astype(jnp.float32)


def _edge_kernel_body(rs_ref, rd_ref, eh_ref, t_ref, et_ref,
                      wq_ref, wk_ref, wv_ref, gs_ref, bs_ref, gd_ref, bd_ref,
                      out_ref):
    rs = rs_ref[...].astype(jnp.float32)
    rd = rd_ref[...].astype(jnp.float32)
    t = t_ref[...]
    et = et_ref[...]
    hs = rs[:, :IN_DIM]
    dia_s = jnp.sin(rs[:, IN_DIM:IN_DIM + TIME_DIM] * t
                    + rs[:, IN_DIM + TIME_DIM:]) * hs[:, :TIME_DIM]
    xs = jnp.concatenate([dia_s, hs[:, TIME_DIM:], eh_ref[...]], axis=1)
    xs = _layer_norm(xs, gs_ref[...], bs_ref[...])
    hd = rd[:, :IN_DIM]
    dia_d = jnp.sin(rd[:, IN_DIM:IN_DIM + TIME_DIM] * t
                    + rd[:, IN_DIM + TIME_DIM:]) * hd[:, :TIME_DIM]
    xd = jnp.concatenate([dia_d, hd[:, TIME_DIM:]], axis=1)
    xd = _layer_norm(xd, gd_ref[...], bd_ref[...])

    q = _typed(xd, et, wq_ref, NUM_RELS)
    k = _typed(xs, et, wk_ref, NUM_RELS)
    v = _typed(xs, et, wv_ref, NUM_RELS)

    attn = jnp.dot(q * k, _head_sum_mat(),
                   preferred_element_type=jnp.float32) * (1.0 / math.sqrt(OUT_DIM))
    ex = jnp.exp(attn)
    vw = v * jnp.dot(ex, _head_bcast_mat(),
                     preferred_element_type=jnp.float32)
    out_ref[...] = jnp.concatenate(
        [ex, jnp.zeros((ex.shape[0], CDIM - OUT_DIM - NUM_HEADS), jnp.float32), vw],
        axis=1)


@jax.jit
def _edge_compute(rows_s, rows_d, edge_h, t_col, et_col, Wq, Wk, Wv,
                  gs, bs, gd, bd):
    grid = (E // EBLK,)
    return pl.pallas_call(
        _edge_kernel_body,
        grid=grid,
        in_specs=[
            pl.BlockSpec((EBLK, ROW), lambda i: (i, 0)),
            pl.BlockSpec((EBLK, ROW), lambda i: (i, 0)),
            pl.BlockSpec((EBLK, E_DIM), lambda i: (i, 0)),
            pl.BlockSpec((EBLK, 1), lambda i: (i, 0)),
            pl.BlockSpec((EBLK, 1), lambda i: (i, 0)),
            pl.BlockSpec((NUM_RELS, IN_DIM, OUT_DIM), lambda i: (0, 0, 0)),
            pl.BlockSpec((NUM_RELS, IN_DIM + E_DIM, OUT_DIM), lambda i: (0, 0, 0)),
            pl.BlockSpec((NUM_RELS, IN_DIM + E_DIM, OUT_DIM), lambda i: (0, 0, 0)),
            pl.BlockSpec((1, IN_DIM + E_DIM), lambda i: (0, 0)),
            pl.BlockSpec((1, IN_DIM + E_DIM), lambda i: (0, 0)),
            pl.BlockSpec((1, IN_DIM), lambda i: (0, 0)),
            pl.BlockSpec((1, IN_DIM), lambda i: (0, 0)),
        ],
        out_specs=pl.BlockSpec((EBLK, CDIM), lambda i: (i, 0)),
        out_shape=jax.ShapeDtypeStruct((E, CDIM), jnp.float32),
    )(rows_s, rows_d, edge_h, t_col, et_col, Wq, Wk, Wv, gs, bs, gd, bd)


NBLK = 1000


def _epi_kernel_body(p_ref, srch_ref, nt_ref, hb_ref, wa_ref, skip_ref, out_ref):
    s = p_ref[0] + p_ref[1]
    den = s[:, :NUM_HEADS]
    num = s[:, NUM_HEADS + (CDIM - OUT_DIM - NUM_HEADS):]
    den = jnp.where(den > 0.0, den, 1.0)
    h = num / jnp.dot(den, _head_bcast_mat(), preferred_element_type=jnp.float32)
    nt = nt_ref[...]
    onehot = (nt == lax.broadcasted_iota(jnp.int32, (1, NUM_NTYPES), 1)
              ).astype(jnp.float32)
    h = h + jnp.dot(onehot, hb_ref[...], preferred_element_type=jnp.float32)
    h2 = _typed(h, nt, wa_ref, NUM_NTYPES)
    al = jax.nn.sigmoid(jnp.dot(onehot, skip_ref[...],
                                preferred_element_type=jnp.float32))
    out_ref[...] = h2 * al + srch_ref[...] * (1.0 - al)


@jax.jit
def _epilogue(partials, src_h, nt_col, h_bias, Wa, skip_col):
    grid = (N // NBLK,)
    return pl.pallas_call(
        _epi_kernel_body,
        grid=grid,
        in_specs=[
            pl.BlockSpec((NC, NBLK, CDIM), lambda i: (0, i, 0)),
            pl.BlockSpec((NBLK, IN_DIM), lambda i: (i, 0)),
            pl.BlockSpec((NBLK, 1), lambda i: (i, 0)),
            pl.BlockSpec((NUM_NTYPES, OUT_DIM), lambda i: (0, 0)),
            pl.BlockSpec((NUM_NTYPES, OUT_DIM, OUT_DIM), lambda i: (0, 0, 0)),
            pl.BlockSpec((NUM_NTYPES, 1), lambda i: (0, 0)),
        ],
        out_specs=pl.BlockSpec((NBLK, OUT_DIM), lambda i: (i, 0)),
        out_shape=jax.ShapeDtypeStruct((N, OUT_DIM), jnp.float32),
    )(partials, src_h, nt_col, h_bias, Wa, skip_col)


def kernel(src_h, src_tw, src_tb, edge_h, edge_date, Wq, Wk, Wv, Wa,
           src_ln_g, src_ln_b, dst_ln_g, dst_ln_b, h_bias, skip,
           edge_index, edge_etype, dst_ntype):
    src = edge_index[0]
    dst = edge_index[1]
    table = jnp.concatenate([src_h, src_tw, src_tb], axis=1).astype(jnp.bfloat16)
    rows_s, rows_d = _gather(table, src, dst)
    contrib = _edge_compute(
        rows_s, rows_d, edge_h,
        edge_date.reshape(E, 1), edge_etype.reshape(E, 1),
        Wq, Wk, Wv,
        src_ln_g.reshape(1, -1), src_ln_b.reshape(1, -1),
        dst_ln_g.reshape(1, -1), dst_ln_b.reshape(1, -1))
    partials = _scatter(contrib, dst, jnp.zeros((N, CDIM), jnp.float32))
    return _epilogue(partials, src_h, dst_ntype.reshape(N, 1),
                     h_bias, Wa, skip.reshape(NUM_NTYPES, 1))


# speedup vs baseline: 1.0405x; 1.0175x over previous
import functools
import math

import jax
import jax.numpy as jnp
from jax import lax
from jax.experimental import pallas as pl
from jax.experimental.pallas import tpu as pltpu
from jax.experimental.pallas import tpu_sc as plsc

N = 10000
E = 160000
IN_DIM = 128
OUT_DIM = 128
E_DIM = 16
TIME_DIM = 32
NUM_HEADS = 8
HEAD = OUT_DIM // NUM_HEADS
NUM_RELS = 8
NUM_NTYPES = 4
ROW = IN_DIM + 2 * TIME_DIM
CDIM = 144

NC = 2
NS = 16
NW = NC * NS
EPW = E // NW
GCHUNK = 200
NCHUNKS = EPW // GCHUNK
NPT = N // NS

@functools.cache
def _sc_mesh():
    return plsc.VectorSubcoreMesh(core_axis_name="c", subcore_axis_name="s",
                                  num_cores=NC, num_subcores=NS)


def _gather_body(table_hbm, srcidx_hbm, dstidx_hbm, out_s_hbm, out_d_hbm,
                 idx_v, rows_v, sem):
    cid = lax.axis_index("c")
    sid = lax.axis_index("s")
    base = (cid * NS + sid) * EPW

    def step(j, carry):
        off = base + j * GCHUNK
        pltpu.sync_copy(srcidx_hbm.at[pl.ds(off, GCHUNK)], idx_v)
        pltpu.async_copy(table_hbm.at[idx_v], rows_v, sem).wait()
        pltpu.sync_copy(rows_v, out_s_hbm.at[pl.ds(off, GCHUNK)])
        pltpu.sync_copy(dstidx_hbm.at[pl.ds(off, GCHUNK)], idx_v)
        pltpu.async_copy(table_hbm.at[idx_v], rows_v, sem).wait()
        pltpu.sync_copy(rows_v, out_d_hbm.at[pl.ds(off, GCHUNK)])
        return carry

    lax.fori_loop(0, NCHUNKS, step, 0)


@jax.jit
def _gather(table, src, dst):
    k = pl.kernel(
        _gather_body,
        out_type=(jax.ShapeDtypeStruct((E, ROW), jnp.float32),
                  jax.ShapeDtypeStruct((E, ROW), jnp.float32)),
        mesh=_sc_mesh(),
        scratch_types=[
            pltpu.VMEM((GCHUNK,), jnp.int32),
            pltpu.VMEM((GCHUNK, ROW), jnp.float32),
            pltpu.SemaphoreType.DMA,
        ],
        compiler_params=pltpu.CompilerParams(use_tc_tiling_on_sc=False),
    )
    return k(table, src, dst)


def _scatter_body(contrib_hbm, dstidx_hbm, zeros_hbm, out_hbm,
                  idx_v, rows_v, acc_sh):
    cid = lax.axis_index("c")
    sid = lax.axis_index("s")
    pltpu.sync_copy(zeros_hbm.at[pl.ds(sid * NPT, NPT)],
                    acc_sh.at[pl.ds(sid * NPT, NPT)])
    plsc.subcore_barrier()
    base = (cid * NS + sid) * EPW

    def step(j, carry):
        off = base + j * GCHUNK
        pltpu.sync_copy(dstidx_hbm.at[pl.ds(off, GCHUNK)], idx_v)
        pltpu.sync_copy(contrib_hbm.at[pl.ds(off, GCHUNK)], rows_v)
        pltpu.sync_copy(rows_v, acc_sh.at[idx_v], add=True)
        return carry

    lax.fori_loop(0, NCHUNKS, step, 0)
    plsc.subcore_barrier()
    pltpu.sync_copy(acc_sh.at[pl.ds(sid * NPT, NPT)],
                    out_hbm.at[cid, pl.ds(sid * NPT, NPT)])


@jax.jit
def _scatter(contrib, dst, zeros):
    k = pl.kernel(
        _scatter_body,
        out_type=jax.ShapeDtypeStruct((NC, N, CDIM), jnp.float32),
        mesh=_sc_mesh(),
        scratch_types=[
            pltpu.VMEM((GCHUNK,), jnp.int32),
            pltpu.VMEM((GCHUNK, CDIM), jnp.float32),
            pltpu.VMEM_SHARED((N, CDIM), jnp.float32),
        ],
        compiler_params=pltpu.CompilerParams(use_tc_tiling_on_sc=False),
    )
    return k(contrib, dst, zeros)


EBLK = 2000


def _layer_norm(x, g, b, eps=1e-5):
    mu = jnp.mean(x, axis=-1, keepdims=True)
    var = jnp.mean((x - mu) * (x - mu), axis=-1, keepdims=True)
    return (x - mu) * jax.lax.rsqrt(var + eps) * g + b


def _typed(x, et, w_ref, nrel, dtype=jnp.float32):
    acc = jnp.zeros((x.shape[0], OUT_DIM), jnp.float32)
    xc = x.astype(dtype)
    for r in range(nrel):
        m = (et == r).astype(jnp.float32)
        acc = acc + m * jnp.dot(xc, w_ref[r].astype(dtype),
                                preferred_element_type=jnp.float32)
    return acc


def _head_sum_mat():
    i = lax.broadcasted_iota(jnp.int32, (OUT_DIM, NUM_HEADS), 0)
    h = lax.broadcasted_iota(jnp.int32, (OUT_DIM, NUM_HEADS), 1)
    return (i // HEAD == h).astype(jnp.float32)


def _head_bcast_mat():
    h = lax.broadcasted_iota(jnp.int32, (NUM_HEADS, OUT_DIM), 0)
    i = lax.broadcasted_iota(jnp.int32, (NUM_HEADS, OUT_DIM), 1)
    return (i // HEAD == h).astype(jnp.float32)


def _edge_kernel_body(rs_ref, rd_ref, eh_ref, t_ref, et_ref,
                      wq_ref, wk_ref, wv_ref, gs_ref, bs_ref, gd_ref, bd_ref,
                      out_ref):
    rs = rs_ref[...]
    rd = rd_ref[...]
    t = t_ref[...]
    et = et_ref[...]
    hs = rs[:, :IN_DIM]
    dia_s = jnp.sin(rs[:, IN_DIM:IN_DIM + TIME_DIM] * t
                    + rs[:, IN_DIM + TIME_DIM:]) * hs[:, :TIME_DIM]
    xs = jnp.concatenate([dia_s, hs[:, TIME_DIM:], eh_ref[...]], axis=1)
    xs = _layer_norm(xs, gs_ref[...], bs_ref[...])
    hd = rd[:, :IN_DIM]
    dia_d = jnp.sin(rd[:, IN_DIM:IN_DIM + TIME_DIM] * t
                    + rd[:, IN_DIM + TIME_DIM:]) * hd[:, :TIME_DIM]
    xd = jnp.concatenate([dia_d, hd[:, TIME_DIM:]], axis=1)
    xd = _layer_norm(xd, gd_ref[...], bd_ref[...])

    q = _typed(xd, et, wq_ref, NUM_RELS)
    k = _typed(xs, et, wk_ref, NUM_RELS)
    v = _typed(xs, et, wv_ref, NUM_RELS)

    attn = jnp.dot(q * k, _head_sum_mat(),
                   preferred_element_type=jnp.float32) * (1.0 / math.sqrt(OUT_DIM))
    ex = jnp.exp(attn)
    vw = v * jnp.dot(ex, _head_bcast_mat(),
                     preferred_element_type=jnp.float32)
    out_ref[...] = jnp.concatenate(
        [ex, jnp.zeros((ex.shape[0], CDIM - OUT_DIM - NUM_HEADS), jnp.float32), vw],
        axis=1)


@jax.jit
def _edge_compute(rows_s, rows_d, edge_h, t_col, et_col, Wq, Wk, Wv,
                  gs, bs, gd, bd):
    grid = (E // EBLK,)
    return pl.pallas_call(
        _edge_kernel_body,
        grid=grid,
        in_specs=[
            pl.BlockSpec((EBLK, ROW), lambda i: (i, 0)),
            pl.BlockSpec((EBLK, ROW), lambda i: (i, 0)),
            pl.BlockSpec((EBLK, E_DIM), lambda i: (i, 0)),
            pl.BlockSpec((EBLK, 1), lambda i: (i, 0)),
            pl.BlockSpec((EBLK, 1), lambda i: (i, 0)),
            pl.BlockSpec((NUM_RELS, IN_DIM, OUT_DIM), lambda i: (0, 0, 0)),
            pl.BlockSpec((NUM_RELS, IN_DIM + E_DIM, OUT_DIM), lambda i: (0, 0, 0)),
            pl.BlockSpec((NUM_RELS, IN_DIM + E_DIM, OUT_DIM), lambda i: (0, 0, 0)),
            pl.BlockSpec((1, IN_DIM + E_DIM), lambda i: (0, 0)),
            pl.BlockSpec((1, IN_DIM + E_DIM), lambda i: (0, 0)),
            pl.BlockSpec((1, IN_DIM), lambda i: (0, 0)),
            pl.BlockSpec((1, IN_DIM), lambda i: (0, 0)),
        ],
        out_specs=pl.BlockSpec((EBLK, CDIM), lambda i: (i, 0)),
        out_shape=jax.ShapeDtypeStruct((E, CDIM), jnp.float32),
    )(rows_s, rows_d, edge_h, t_col, et_col, Wq, Wk, Wv, gs, bs, gd, bd)


NBLK = 1000


def _epi_kernel_body(p_ref, srch_ref, nt_ref, hb_ref, wa_ref, skip_ref, out_ref):
    s = p_ref[0] + p_ref[1]
    den = s[:, :NUM_HEADS]
    num = s[:, NUM_HEADS + (CDIM - OUT_DIM - NUM_HEADS):]
    den = jnp.where(den > 0.0, den, 1.0)
    h = num / jnp.dot(den, _head_bcast_mat(), preferred_element_type=jnp.float32)
    nt = nt_ref[...]
    onehot = (nt == lax.broadcasted_iota(jnp.int32, (1, NUM_NTYPES), 1)
              ).astype(jnp.float32)
    h = h + jnp.dot(onehot, hb_ref[...], preferred_element_type=jnp.float32)
    h2 = _typed(h, nt, wa_ref, NUM_NTYPES)
    al = jax.nn.sigmoid(jnp.dot(onehot, skip_ref[...],
                                preferred_element_type=jnp.float32))
    out_ref[...] = h2 * al + srch_ref[...] * (1.0 - al)


@jax.jit
def _epilogue(partials, src_h, nt_col, h_bias, Wa, skip_col):
    grid = (N // NBLK,)
    return pl.pallas_call(
        _epi_kernel_body,
        grid=grid,
        in_specs=[
            pl.BlockSpec((NC, NBLK, CDIM), lambda i: (0, i, 0)),
            pl.BlockSpec((NBLK, IN_DIM), lambda i: (i, 0)),
            pl.BlockSpec((NBLK, 1), lambda i: (i, 0)),
            pl.BlockSpec((NUM_NTYPES, OUT_DIM), lambda i: (0, 0)),
            pl.BlockSpec((NUM_NTYPES, OUT_DIM, OUT_DIM), lambda i: (0, 0, 0)),
            pl.BlockSpec((NUM_NTYPES, 1), lambda i: (0, 0)),
        ],
        out_specs=pl.BlockSpec((NBLK, OUT_DIM), lambda i: (i, 0)),
        out_shape=jax.ShapeDtypeStruct((N, OUT_DIM), jnp.float32),
    )(partials, src_h, nt_col, h_bias, Wa, skip_col)


def kernel(src_h, src_tw, src_tb, edge_h, edge_date, Wq, Wk, Wv, Wa,
           src_ln_g, src_ln_b, dst_ln_g, dst_ln_b, h_bias, skip,
           edge_index, edge_etype, dst_ntype):
    src = edge_index[0]
    dst = edge_index[1]
    table = jnp.concatenate([src_h, src_tw, src_tb], axis=1)
    rows_s, rows_d = _gather(table, src, dst)
    contrib = _edge_compute(
        rows_s, rows_d, edge_h,
        edge_date.reshape(E, 1), edge_etype.reshape(E, 1),
        Wq, Wk, Wv,
        src_ln_g.reshape(1, -1), src_ln_b.reshape(1, -1),
        dst_ln_g.reshape(1, -1), dst_ln_b.reshape(1, -1))
    partials = _scatter(contrib, dst, jnp.zeros((N, CDIM), jnp.float32))
    return _epilogue(partials, src_h, dst_ntype.reshape(N, 1),
                     h_bias, Wa, skip.reshape(NUM_NTYPES, 1))
